# Initial kernel scaffold; baseline (speedup 1.0000x reference)
import jax
import jax.numpy as jnp
from jax.experimental import pallas as pl


def kernel(cl, re, anc):
    raise NotImplementedError("write your pallas kernel here")


# speedup vs baseline: 76.1181x; 76.1181x over previous
import jax
import jax.numpy as jnp
from jax.experimental import pallas as pl
from jax.experimental.pallas import tpu as pltpu

B = 4
N = 5000
T = 128
NB = 40
NP = NB * T
K_OUT = 100
IM_W = 511.0
THR = 0.5

_DOT_KW = dict(precision=jax.lax.Precision.HIGHEST,
               preferred_element_type=jnp.float32)


def _nms_body(anc_r, re_r, ss_r, out_ref, rows_ref, keep_ref, rank_ref,
              a_ref, cand_ref):
    f32 = jnp.float32

    x1r = jnp.maximum(anc_r[:, 0:1, :] - re_r[:, 0:1, :], 0.0)
    y1r = jnp.maximum(anc_r[:, 1:2, :] - re_r[:, 1:2, :], 0.0)
    x2r = jnp.minimum(anc_r[:, 2:3, :] + re_r[:, 2:3, :], IM_W)
    y2r = jnp.minimum(anc_r[:, 3:4, :] + re_r[:, 3:4, :], IM_W)
    arear = jnp.maximum(x2r - x1r, 0.0) * jnp.maximum(y2r - y1r, 0.0)
    rows_ref[:, 0:1, :] = x1r
    rows_ref[:, 1:2, :] = y1r
    rows_ref[:, 2:3, :] = x2r
    rows_ref[:, 3:4, :] = y2r
    rows_ref[:, 4:5, :] = arear
    rows_ref[:, 5:6, :] = ss_r[:, :, :]

    iota_sub = jax.lax.broadcasted_iota(jnp.int32, (1, T, 1), 1)
    iota_lane = jax.lax.broadcasted_iota(jnp.int32, (1, 1, T), 2)
    diag2 = (jax.lax.broadcasted_iota(jnp.int32, (T, T), 0)
             == jax.lax.broadcasted_iota(jnp.int32, (T, T), 1))
    ident = jnp.where(diag2, 1.0, 0.0).astype(f32)

    def _iou_hits(ca, cb):
        x1a, y1a, x2a, y2a, aa = ca
        x1b, y1b, x2b, y2b, ab = cb
        xx1 = jnp.maximum(x1a, x1b)
        yy1 = jnp.maximum(y1a, y1b)
        xx2 = jnp.minimum(x2a, x2b)
        yy2 = jnp.minimum(y2a, y2b)
        inter = jnp.maximum(xx2 - xx1, 0.0) * jnp.maximum(yy2 - yy1, 0.0)
        den = ab + aa - inter + 1e-9
        iou = inter / den
        return jnp.where(iou > THR, 1.0, 0.0).astype(f32)

    def _row5(bi, c):
        off = pl.multiple_of(c * T, T)
        ch = rows_ref[bi, 0:5, pl.ds(off, T)]
        return tuple(ch[k:k + 1, :] for k in range(5))

    def _cand5(bi):
        return tuple(cand_ref[bi, :, k:k + 1] for k in range(5))

    def block_body(b, carry):
        valid = (b * T + iota_sub[0]) < N
        sup0 = jnp.where(valid, 0.0, 1.0).astype(f32)

        for bi in range(B):
            off = pl.multiple_of(b * T, T)
            blk5 = rows_ref[bi, 0:5, pl.ds(off, T)]
            cand = jax.lax.dot_general(
                ident, blk5, (((1,), (1,)), ((), ())), **_DOT_KW)
            cand_ref[bi, :, 0:5] = cand

        for bi in range(B):
            ca = _cand5(bi)

            def chunk_body(c, sup, bi=bi, ca=ca):
                hits = _iou_hits(ca, _row5(bi, c))
                keepch = keep_ref[bi, pl.ds(c, 1), :]
                hits = hits * keepch
                return jnp.maximum(sup, jnp.max(hits, axis=-1, keepdims=True))

            sup_cand = jax.lax.fori_loop(0, b, chunk_body, sup0)

            a_bb = _iou_hits(ca, _row5(bi, b))
            a_ref[bi, :, :] = jnp.where(
                diag2, jnp.broadcast_to(sup_cand, (T, T)), a_bb)

        def step(t, keep_row):
            arow = a_ref[:, pl.ds(t, 1), :]
            oh = jnp.where(iota_lane == t, 1.0, 0.0).astype(f32)
            s = jnp.sum(arow * (keep_row + oh), axis=-1, keepdims=True)
            kt = jnp.where(s > 0.0, 0.0, 1.0)
            return keep_row + oh * kt

        keep_row = jax.lax.fori_loop(0, T, step, jnp.zeros((B, 1, T), f32))
        keep_ref[:, pl.ds(b, 1), :] = keep_row
        return carry

    jax.lax.fori_loop(0, NB, block_body, 0)

    def rank_body(c, run):
        kc = keep_ref[:, pl.ds(c, 1), :]
        x = kc
        sh = 1
        while sh < T:
            x = x + jnp.concatenate(
                [jnp.zeros((B, 1, sh), f32), x[:, :, :T - sh]], axis=-1)
            sh *= 2
        rank_ref[:, pl.ds(c, 1), :] = x + run
        return run + jnp.sum(kc, axis=-1, keepdims=True)

    num = jax.lax.fori_loop(0, NB, rank_body, jnp.zeros((B, 1, 1), f32))

    out_ref[:, :, :] = jnp.zeros((B, T, 5), f32)

    def sel_body(c, carry):
        targets = jnp.minimum(iota_sub.astype(f32) + 1.0, num)
        rch = rank_ref[:, pl.ds(c, 1), :]
        kch = keep_ref[:, pl.ds(c, 1), :]
        mc = jnp.where((rch == targets) & (kch > 0.0), 1.0, 0.0).astype(f32)
        off = pl.multiple_of(c * T, T)
        for c5, src in enumerate([0, 1, 2, 3, 5]):
            row = rows_ref[:, src:src + 1, pl.ds(off, T)]
            contrib = jnp.sum(mc * row, axis=-1, keepdims=True)
            out_ref[:, :, c5:c5 + 1] += contrib
        return carry

    jax.lax.fori_loop(0, NB, sel_body, 0)


_SCRATCH = [
    pltpu.VMEM((B, 6, NP), jnp.float32),
    pltpu.VMEM((B, NB, T), jnp.float32),
    pltpu.VMEM((B, NB, T), jnp.float32),
    pltpu.VMEM((B, T, T), jnp.float32),
    pltpu.VMEM((B, T, 8), jnp.float32),
]


@jax.jit
def _run(anc_r, re_r, ss_r):
    return pl.pallas_call(
        _nms_body,
        out_shape=jax.ShapeDtypeStruct((B, T, 5), jnp.float32),
        scratch_shapes=_SCRATCH,
    )(anc_r, re_r, ss_r)


def _prepare(cl, re, anc):
    ff = jnp.exp(cl[..., 0]) / (jnp.exp(cl[..., 0]) + jnp.exp(cl[..., 1]))
    order = jnp.argsort(-ff, axis=1)
    ss = jnp.take_along_axis(ff, order, axis=1)
    re_s = jnp.take_along_axis(re, order[..., None], axis=1)
    anc_b = jnp.broadcast_to(anc, (B,) + anc.shape[1:])
    anc_s = jnp.take_along_axis(anc_b, order[..., None], axis=1)

    pad = NP - N
    anc_r = jnp.pad(anc_s, ((0, 0), (0, pad), (0, 0))).transpose(0, 2, 1)
    re_r = jnp.pad(re_s, ((0, 0), (0, pad), (0, 0))).transpose(0, 2, 1)
    ss_r = jnp.pad(ss, ((0, 0), (0, pad)))[:, None, :]
    return anc_r, re_r, ss_r


def kernel(cl, re, anc):
    out = _run(*_prepare(cl, re, anc))
    return out[:, :K_OUT, 0:4], out[:, :K_OUT, 4]


# speedup vs baseline: 689.2921x; 9.0556x over previous
import jax
import jax.numpy as jnp
from jax.experimental import pallas as pl
from jax.experimental.pallas import tpu as pltpu

B = 4
N = 5000
T = 128
NB = 40
NP = NB * T
K_OUT = 100
IM_W = 511.0
THR = 0.5

_DOT_KW = dict(precision=jax.lax.Precision.HIGHEST,
               preferred_element_type=jnp.float32)


def _nms_body(anc_r, re_r, ss_r, out_ref, rows_ref, keep_ref, rank_ref,
              a_ref, cand_ref):
    f32 = jnp.float32

    x1r = jnp.maximum(anc_r[:, 0:1, :] - re_r[:, 0:1, :], 0.0)
    y1r = jnp.maximum(anc_r[:, 1:2, :] - re_r[:, 1:2, :], 0.0)
    x2r = jnp.minimum(anc_r[:, 2:3, :] + re_r[:, 2:3, :], IM_W)
    y2r = jnp.minimum(anc_r[:, 3:4, :] + re_r[:, 3:4, :], IM_W)
    arear = jnp.maximum(x2r - x1r, 0.0) * jnp.maximum(y2r - y1r, 0.0)
    rows_ref[:, 0:1, :] = x1r
    rows_ref[:, 1:2, :] = y1r
    rows_ref[:, 2:3, :] = x2r
    rows_ref[:, 3:4, :] = y2r
    rows_ref[:, 4:5, :] = arear
    rows_ref[:, 5:6, :] = ss_r[:, :, :]

    iota_sub = jax.lax.broadcasted_iota(jnp.int32, (1, T, 1), 1)
    iota_lane = jax.lax.broadcasted_iota(jnp.int32, (1, 1, T), 2)
    diag2 = (jax.lax.broadcasted_iota(jnp.int32, (T, T), 0)
             == jax.lax.broadcasted_iota(jnp.int32, (T, T), 1))
    ident = jnp.where(diag2, 1.0, 0.0).astype(f32)

    def _iou_hits(ca, cb):
        x1a, y1a, x2a, y2a, aa = ca
        x1b, y1b, x2b, y2b, ab = cb
        xx1 = jnp.maximum(x1a, x1b)
        yy1 = jnp.maximum(y1a, y1b)
        xx2 = jnp.minimum(x2a, x2b)
        yy2 = jnp.minimum(y2a, y2b)
        inter = jnp.maximum(xx2 - xx1, 0.0) * jnp.maximum(yy2 - yy1, 0.0)
        den = ab + aa - inter + 1e-9
        iou = inter / den
        return jnp.where(iou > THR, 1.0, 0.0).astype(f32)

    def _row5(bi, c):
        off = pl.multiple_of(c * T, T)
        ch = rows_ref[bi, 0:5, pl.ds(off, T)]
        return tuple(ch[k:k + 1, :] for k in range(5))

    def _cand5(bi):
        return tuple(cand_ref[bi, :, k:k + 1] for k in range(5))

    keep_ref[:, :, :] = jnp.zeros((B, NB, T), f32)

    def block_cond(state):
        b, cnt = state
        return jnp.logical_and(b < NB, jnp.min(cnt) < K_OUT)

    def block_body(state):
        b, cnt = state
        valid = (b * T + iota_sub[0]) < N
        sup0 = jnp.where(valid, 0.0, 1.0).astype(f32)

        for bi in range(B):
            off = pl.multiple_of(b * T, T)
            blk5 = rows_ref[bi, 0:5, pl.ds(off, T)]
            cand = jax.lax.dot_general(
                ident, blk5, (((1,), (1,)), ((), ())), **_DOT_KW)
            cand_ref[bi, :, 0:5] = cand

        for bi in range(B):
            ca = _cand5(bi)

            def chunk_body(c, sup, bi=bi, ca=ca):
                hits = _iou_hits(ca, _row5(bi, c))
                keepch = keep_ref[bi, pl.ds(c, 1), :]
                hits = hits * keepch
                return jnp.maximum(sup, jnp.max(hits, axis=-1, keepdims=True))

            sup_cand = jax.lax.fori_loop(0, b, chunk_body, sup0)

            a_bb = _iou_hits(ca, _row5(bi, b))
            a_ref[bi, :, :] = jnp.where(
                diag2, jnp.broadcast_to(sup_cand, (T, T)), a_bb)

        def step(t, keep_row):
            arow = a_ref[:, pl.ds(t, 1), :]
            oh = jnp.where(iota_lane == t, 1.0, 0.0).astype(f32)
            s = jnp.sum(arow * (keep_row + oh), axis=-1, keepdims=True)
            kt = jnp.where(s > 0.0, 0.0, 1.0)
            return keep_row + oh * kt

        keep_row = jax.lax.fori_loop(0, T, step, jnp.zeros((B, 1, T), f32))
        keep_ref[:, pl.ds(b, 1), :] = keep_row
        return (b + 1, cnt + jnp.sum(keep_row, axis=-1, keepdims=True))

    jax.lax.while_loop(block_cond, block_body,
                       (jnp.int32(0), jnp.zeros((B, 1, 1), f32)))

    def rank_body(c, run):
        kc = keep_ref[:, pl.ds(c, 1), :]
        x = kc
        sh = 1
        while sh < T:
            x = x + jnp.concatenate(
                [jnp.zeros((B, 1, sh), f32), x[:, :, :T - sh]], axis=-1)
            sh *= 2
        rank_ref[:, pl.ds(c, 1), :] = x + run
        return run + jnp.sum(kc, axis=-1, keepdims=True)

    num = jax.lax.fori_loop(0, NB, rank_body, jnp.zeros((B, 1, 1), f32))

    out_ref[:, :, :] = jnp.zeros((B, T, 5), f32)

    def sel_body(c, carry):
        targets = jnp.minimum(iota_sub.astype(f32) + 1.0, num)
        rch = rank_ref[:, pl.ds(c, 1), :]
        kch = keep_ref[:, pl.ds(c, 1), :]
        mc = jnp.where((rch == targets) & (kch > 0.0), 1.0, 0.0).astype(f32)
        off = pl.multiple_of(c * T, T)
        for c5, src in enumerate([0, 1, 2, 3, 5]):
            row = rows_ref[:, src:src + 1, pl.ds(off, T)]
            contrib = jnp.sum(mc * row, axis=-1, keepdims=True)
            out_ref[:, :, c5:c5 + 1] += contrib
        return carry

    jax.lax.fori_loop(0, NB, sel_body, 0)


_SCRATCH = [
    pltpu.VMEM((B, 6, NP), jnp.float32),
    pltpu.VMEM((B, NB, T), jnp.float32),
    pltpu.VMEM((B, NB, T), jnp.float32),
    pltpu.VMEM((B, T, T), jnp.float32),
    pltpu.VMEM((B, T, 8), jnp.float32),
]


@jax.jit
def _run(anc_r, re_r, ss_r):
    return pl.pallas_call(
        _nms_body,
        out_shape=jax.ShapeDtypeStruct((B, T, 5), jnp.float32),
        scratch_shapes=_SCRATCH,
    )(anc_r, re_r, ss_r)


def _prepare(cl, re, anc):
    ff = jnp.exp(cl[..., 0]) / (jnp.exp(cl[..., 0]) + jnp.exp(cl[..., 1]))
    order = jnp.argsort(-ff, axis=1)
    ss = jnp.take_along_axis(ff, order, axis=1)
    re_s = jnp.take_along_axis(re, order[..., None], axis=1)
    anc_b = jnp.broadcast_to(anc, (B,) + anc.shape[1:])
    anc_s = jnp.take_along_axis(anc_b, order[..., None], axis=1)

    pad = NP - N
    anc_r = jnp.pad(anc_s, ((0, 0), (0, pad), (0, 0))).transpose(0, 2, 1)
    re_r = jnp.pad(re_s, ((0, 0), (0, pad), (0, 0))).transpose(0, 2, 1)
    ss_r = jnp.pad(ss, ((0, 0), (0, pad)))[:, None, :]
    return anc_r, re_r, ss_r


def kernel(cl, re, anc):
    out = _run(*_prepare(cl, re, anc))
    return out[:, :K_OUT, 0:4], out[:, :K_OUT, 4]


# speedup vs baseline: 807.5542x; 1.1716x over previous
import functools

import jax
import jax.numpy as jnp
from jax import lax
from jax.experimental import pallas as pl
from jax.experimental.pallas import tpu as pltpu
from jax.experimental.pallas import tpu_sc as plsc

B = 4
N = 5000
T = 128
NB = 40
NP = NB * T
K_OUT = 100
IM_W = 511.0
THR = 0.5

_DOT_KW = dict(precision=jax.lax.Precision.HIGHEST,
               preferred_element_type=jnp.float32)


def _nms_body(anc_r, re_r, ss_r, data_out, pos_out,
              keep_ref, rank_ref, rows_ref, a_ref, cand_ref):
    f32 = jnp.float32

    x1r = jnp.maximum(anc_r[:, 0:1, :] - re_r[:, 0:1, :], 0.0)
    y1r = jnp.maximum(anc_r[:, 1:2, :] - re_r[:, 1:2, :], 0.0)
    x2r = jnp.minimum(anc_r[:, 2:3, :] + re_r[:, 2:3, :], IM_W)
    y2r = jnp.minimum(anc_r[:, 3:4, :] + re_r[:, 3:4, :], IM_W)
    arear = jnp.maximum(x2r - x1r, 0.0) * jnp.maximum(y2r - y1r, 0.0)
    rows_ref[:, 0:1, :] = x1r
    rows_ref[:, 1:2, :] = y1r
    rows_ref[:, 2:3, :] = x2r
    rows_ref[:, 3:4, :] = y2r
    rows_ref[:, 4:5, :] = arear
    rows_ref[:, 5:6, :] = ss_r[:, :, :]

    iota_sub = jax.lax.broadcasted_iota(jnp.int32, (1, T, 1), 1)
    iota_lane = jax.lax.broadcasted_iota(jnp.int32, (1, 1, T), 2)
    diag2 = (jax.lax.broadcasted_iota(jnp.int32, (T, T), 0)
             == jax.lax.broadcasted_iota(jnp.int32, (T, T), 1))
    ident = jnp.where(diag2, 1.0, 0.0).astype(f32)

    def _iou_hits(ca, cb):
        x1a, y1a, x2a, y2a, aa = ca
        x1b, y1b, x2b, y2b, ab = cb
        xx1 = jnp.maximum(x1a, x1b)
        yy1 = jnp.maximum(y1a, y1b)
        xx2 = jnp.minimum(x2a, x2b)
        yy2 = jnp.minimum(y2a, y2b)
        inter = jnp.maximum(xx2 - xx1, 0.0) * jnp.maximum(yy2 - yy1, 0.0)
        den = ab + aa - inter + 1e-9
        iou = inter / den
        return jnp.where(iou > THR, 1.0, 0.0).astype(f32)

    def _row5(bi, c):
        off = pl.multiple_of(c * T, T)
        ch = rows_ref[bi, 0:5, pl.ds(off, T)]
        return tuple(ch[k:k + 1, :] for k in range(5))

    def _cand5(bi):
        return tuple(cand_ref[bi, :, k:k + 1] for k in range(5))

    keep_ref[:, :, :] = jnp.zeros((B, NB, T), f32)

    def block_cond(state):
        b, cnt = state
        return jnp.logical_and(b < NB, jnp.min(cnt) < K_OUT)

    def block_body(state):
        b, cnt = state
        valid = (b * T + iota_sub[0]) < N
        sup0 = jnp.where(valid, 0.0, 1.0).astype(f32)

        for bi in range(B):
            off = pl.multiple_of(b * T, T)
            blk6 = rows_ref[bi, 0:6, pl.ds(off, T)]
            cand = jax.lax.dot_general(
                ident, blk6, (((1,), (1,)), ((), ())), **_DOT_KW)
            cand_ref[bi, :, 0:6] = cand
            data_out[bi, pl.ds(off, T), 0:6] = cand

        for bi in range(B):
            ca = _cand5(bi)

            def chunk_body(c, sup, bi=bi, ca=ca):
                hits = _iou_hits(ca, _row5(bi, c))
                keepch = keep_ref[bi, pl.ds(c, 1), :]
                hits = hits * keepch
                return jnp.maximum(sup, jnp.max(hits, axis=-1, keepdims=True))

            sup_cand = jax.lax.fori_loop(0, b, chunk_body, sup0)

            a_bb = _iou_hits(ca, _row5(bi, b))
            a_ref[bi, :, :] = jnp.where(
                diag2, jnp.broadcast_to(sup_cand, (T, T)), a_bb)

        def step(t, keep_row):
            arow = a_ref[:, pl.ds(t, 1), :]
            oh = jnp.where(iota_lane == t, 1.0, 0.0).astype(f32)
            s = jnp.sum(arow * (keep_row + oh), axis=-1, keepdims=True)
            kt = jnp.where(s > 0.0, 0.0, 1.0)
            return keep_row + oh * kt

        keep_row = jax.lax.fori_loop(0, T, step, jnp.zeros((B, 1, T), f32))
        keep_ref[:, pl.ds(b, 1), :] = keep_row
        return (b + 1, cnt + jnp.sum(keep_row, axis=-1, keepdims=True))

    bfin, _ = jax.lax.while_loop(block_cond, block_body,
                                 (jnp.int32(0), jnp.zeros((B, 1, 1), f32)))

    def rank_body(c, run):
        kc = keep_ref[:, pl.ds(c, 1), :]
        x = kc
        sh = 1
        while sh < T:
            x = x + jnp.concatenate(
                [jnp.zeros((B, 1, sh), f32), x[:, :, :T - sh]], axis=-1)
            sh *= 2
        rank_ref[:, pl.ds(c, 1), :] = x + run
        return run + jnp.sum(kc, axis=-1, keepdims=True)

    num = jax.lax.fori_loop(0, bfin, rank_body, jnp.zeros((B, 1, 1), f32))

    targets = jnp.minimum(iota_sub.astype(f32) + 1.0, num)

    def sel_body(c, acc):
        rch = rank_ref[:, pl.ds(c, 1), :]
        kch = keep_ref[:, pl.ds(c, 1), :]
        mc = jnp.where((rch == targets) & (kch > 0.0), 1.0, 0.0).astype(f32)
        jrow = (c * T + iota_lane).astype(f32)
        return acc + jnp.sum(mc * jrow, axis=-1, keepdims=True)

    pos = jax.lax.fori_loop(0, bfin, sel_body, jnp.zeros((B, T, 1), f32))
    pos_out[:, :, :] = pos


@jax.jit
def _run_nms(anc_r, re_r, ss_r):
    return pl.pallas_call(
        _nms_body,
        out_shape=(
            jax.ShapeDtypeStruct((B, NP, 128), jnp.float32),
            jax.ShapeDtypeStruct((B, T, 1), jnp.float32),
        ),
        scratch_shapes=[
            pltpu.VMEM((B, NB, T), jnp.float32),
            pltpu.VMEM((B, NB, T), jnp.float32),
            pltpu.VMEM((B, 6, NP), jnp.float32),
            pltpu.VMEM((B, T, T), jnp.float32),
            pltpu.VMEM((B, T, 8), jnp.float32),
        ],
    )(anc_r, re_r, ss_r)


_SC_GATHER = None


def _get_sc_gather():
    global _SC_GATHER
    if _SC_GATHER is None:
        mesh = plsc.VectorSubcoreMesh(core_axis_name="c",
                                      subcore_axis_name="s")

        @functools.partial(
            pl.kernel, mesh=mesh,
            out_type=jax.ShapeDtypeStruct((B * T, 128), jnp.float32),
            scratch_types=[
                pltpu.VMEM((T,), jnp.int32),
                pltpu.VMEM((T, 128), jnp.float32),
                pltpu.SemaphoreType.DMA,
            ],
        )
        def _sc_gather(table_hbm, idx_hbm, out_hbm, idx_v, buf_v, sem):
            wid = lax.axis_index("s") * 2 + lax.axis_index("c")
            base = (wid % B) * T
            pltpu.sync_copy(idx_hbm.at[pl.ds(base, T)], idx_v)
            pltpu.async_copy(table_hbm.at[idx_v], buf_v, sem).wait()
            pltpu.sync_copy(buf_v, out_hbm.at[pl.ds(base, T)])

        _SC_GATHER = _sc_gather
    return _SC_GATHER


def _prepare(cl, re, anc):
    ff = jnp.exp(cl[..., 0]) / (jnp.exp(cl[..., 0]) + jnp.exp(cl[..., 1]))
    anc_b = jnp.broadcast_to(anc, (B,) + anc.shape[1:])
    ops = (-ff, ff) + tuple(anc_b[..., k] for k in range(4)) \
        + tuple(re[..., k] for k in range(4))
    sops = jax.lax.sort(ops, dimension=1, is_stable=True, num_keys=1)
    ss = sops[1]
    anc_r = jnp.stack(sops[2:6], axis=1)
    re_r = jnp.stack(sops[6:10], axis=1)

    pad = NP - N
    anc_r = jnp.pad(anc_r, ((0, 0), (0, 0), (0, pad)))
    re_r = jnp.pad(re_r, ((0, 0), (0, 0), (0, pad)))
    ss_r = jnp.pad(ss, ((0, 0), (0, pad)))[:, None, :]
    return anc_r, re_r, ss_r


def kernel(cl, re, anc):
    data16, pos = _run_nms(*_prepare(cl, re, anc))
    idx = (pos.reshape(B, T).astype(jnp.int32)
           + (jnp.arange(B, dtype=jnp.int32) * NP)[:, None])
    out = _get_sc_gather()(data16.reshape(B * NP, 128), idx.reshape(B * T))
    out = out.reshape(B, T, 128)
    return out[:, :K_OUT, 0:4], out[:, :K_OUT, 5]
